# baseline (speedup 1.0000x reference)
import jax
import jax.numpy as jnp
from jax.experimental import pallas as pl
from jax.experimental.pallas import tpu as pltpu

VOCAB = 100000
D = 64
NCTX = 20
HID = 128
VB = 8192
NVB = (VOCAB + VB - 1) // VB


def _gather_mlp1_kernel(idx_ref, emb_hbm, w1_ref, b1_ref, out_ref,
                        rows_ref, sem):
    copies = [
        pltpu.make_async_copy(
            emb_hbm.at[pl.ds(idx_ref[k], 1), :],
            rows_ref.at[pl.ds(k, 1), :],
            sem,
        )
        for k in range(NCTX)
    ]
    for c in copies:
        c.start()
    for c in copies:
        c.wait()

    acc = b1_ref[...]
    for k in range(NCTX):
        acc = acc + jnp.dot(rows_ref[pl.ds(k, 1), :],
                            w1_ref[pl.ds(k * D, D), :],
                            preferred_element_type=jnp.float32)
    out_ref[...] = jnp.maximum(acc, 0.0)


def _mlp2_logsoftmax_kernel(h_ref, w2_ref, b2_ref, out_ref,
                            acc_ref, m_ref, s_ref):
    p = pl.program_id(0)
    i = pl.program_id(1)

    @pl.when((p == 0) & (i == 0))
    def _():
        m_ref[0] = -jnp.inf
        s_ref[0] = 0.0

    @pl.when(p == 0)
    def _():
        z = jnp.dot(h_ref[...], w2_ref[...],
                    preferred_element_type=jnp.float32) + b2_ref[...]
        col = i * VB + jax.lax.broadcasted_iota(jnp.int32, (1, VB), 1)
        z = jnp.where(col < VOCAB, z, -jnp.inf)

        m_old = m_ref[0]
        m_new = jnp.maximum(m_old, jnp.max(z))
        s_ref[0] = (s_ref[0] * jnp.exp(m_old - m_new)
                    + jnp.sum(jnp.exp(z - m_new)))
        m_ref[0] = m_new
        acc_ref[pl.ds(i, 1), :] = z

    @pl.when(p == 1)
    def _():
        norm = m_ref[0] + jnp.log(s_ref[0])
        out_ref[...] = acc_ref[pl.ds(i, 1), :] - norm


def kernel(inputs, emb_table, W1, b1, W2, b2):
    idx = inputs.astype(jnp.int32)

    h = pl.pallas_call(
        _gather_mlp1_kernel,
        in_specs=[
            pl.BlockSpec(memory_space=pltpu.SMEM),
            pl.BlockSpec(memory_space=pltpu.MemorySpace.HBM),
            pl.BlockSpec(memory_space=pltpu.VMEM),
            pl.BlockSpec(memory_space=pltpu.VMEM),
        ],
        out_specs=pl.BlockSpec(memory_space=pltpu.VMEM),
        out_shape=jax.ShapeDtypeStruct((1, HID), jnp.float32),
        scratch_shapes=[pltpu.VMEM((NCTX, D), jnp.float32),
                        pltpu.SemaphoreType.DMA],
    )(idx, emb_table, W1, b1.reshape(1, HID))

    log_probs = pl.pallas_call(
        _mlp2_logsoftmax_kernel,
        grid=(2, NVB),
        in_specs=[
            pl.BlockSpec((1, HID), lambda p, i: (0, 0)),
            pl.BlockSpec((HID, VB), lambda p, i: (0, jnp.where(p == 0, i, 0))),
            pl.BlockSpec((1, VB), lambda p, i: (0, jnp.where(p == 0, i, 0))),
        ],
        out_specs=pl.BlockSpec((1, VB),
                               lambda p, i: (0, jnp.where(p == 0, 0, i))),
        out_shape=jax.ShapeDtypeStruct((1, VOCAB), jnp.float32),
        scratch_shapes=[pltpu.VMEM((NVB, VB), jnp.float32),
                        pltpu.SMEM((1,), jnp.float32),
                        pltpu.SMEM((1,), jnp.float32)],
    )(h, W2, b2.reshape(1, VOCAB))

    return log_probs


# speedup vs baseline: 1.0345x; 1.0345x over previous
import jax
import jax.numpy as jnp
from jax.experimental import pallas as pl
from jax.experimental.pallas import tpu as pltpu

VOCAB = 100000
D = 64
NCTX = 20
HID = 128
VB = 8192
NVB = (VOCAB + VB - 1) // VB
G = (NVB + 1) // 2


def _gather_mlp1_kernel(idx_ref, emb_hbm, w1_ref, b1_ref, out_ref,
                        rows_ref, sem):
    copies = [
        pltpu.make_async_copy(
            emb_hbm.at[pl.ds(idx_ref[k], 1), :],
            rows_ref.at[pl.ds(k, 1), :],
            sem,
        )
        for k in range(NCTX)
    ]
    for c in copies:
        c.start()
    for c in copies:
        c.wait()

    acc = b1_ref[...]
    for k in range(NCTX):
        acc = acc + jnp.dot(rows_ref[pl.ds(k, 1), :],
                            w1_ref[pl.ds(k * D, D), :],
                            preferred_element_type=jnp.float32)
    out_ref[...] = jnp.maximum(acc, 0.0)


def _mlp2_logsoftmax_kernel(h_ref, w2a_ref, w2b_ref, b2_ref, out_ref,
                            acc_ref, m_ref, s_ref):
    p = pl.program_id(0)
    i = pl.program_id(1)

    @pl.when((p == 0) & (i == 0))
    def _():
        m_ref[0] = -jnp.inf
        s_ref[0] = 0.0

    @pl.when(p == 0)
    def _():
        h = h_ref[...]
        za = jnp.dot(h, w2a_ref[...],
                     preferred_element_type=jnp.float32) + b2_ref[:, :VB]
        zb = jnp.dot(h, w2b_ref[...],
                     preferred_element_type=jnp.float32) + b2_ref[:, VB:]
        lane = jax.lax.broadcasted_iota(jnp.int32, (1, VB), 1)
        za = jnp.where((2 * i) * VB + lane < VOCAB, za, -jnp.inf)
        zb = jnp.where((2 * i + 1) * VB + lane < VOCAB, zb, -jnp.inf)

        m_old = m_ref[0]
        m_new = jnp.maximum(m_old,
                            jnp.maximum(jnp.max(za), jnp.max(zb)))
        s_ref[0] = (s_ref[0] * jnp.exp(m_old - m_new)
                    + jnp.sum(jnp.exp(za - m_new))
                    + jnp.sum(jnp.exp(zb - m_new)))
        m_ref[0] = m_new
        acc_ref[pl.ds(i, 1), :VB] = za
        acc_ref[pl.ds(i, 1), VB:] = zb

    @pl.when(p == 1)
    def _():
        norm = m_ref[0] + jnp.log(s_ref[0])
        out_ref[...] = acc_ref[pl.ds(i, 1), :] - norm


def kernel(inputs, emb_table, W1, b1, W2, b2):
    idx = inputs.astype(jnp.int32)

    h = pl.pallas_call(
        _gather_mlp1_kernel,
        in_specs=[
            pl.BlockSpec(memory_space=pltpu.SMEM),
            pl.BlockSpec(memory_space=pltpu.MemorySpace.HBM),
            pl.BlockSpec(memory_space=pltpu.VMEM),
            pl.BlockSpec(memory_space=pltpu.VMEM),
        ],
        out_specs=pl.BlockSpec(memory_space=pltpu.VMEM),
        out_shape=jax.ShapeDtypeStruct((1, HID), jnp.float32),
        scratch_shapes=[pltpu.VMEM((NCTX, D), jnp.float32),
                        pltpu.SemaphoreType.DMA],
    )(idx, emb_table, W1, b1.reshape(1, HID))

    log_probs = pl.pallas_call(
        _mlp2_logsoftmax_kernel,
        grid=(2, G),
        in_specs=[
            pl.BlockSpec((1, HID), lambda p, i: (0, 0)),
            pl.BlockSpec((HID, VB),
                         lambda p, i: (0, jnp.where(p == 0, 2 * i, 0))),
            pl.BlockSpec((HID, VB),
                         lambda p, i: (0, jnp.where(
                             p == 0, jnp.minimum(2 * i + 1, NVB - 1), 0))),
            pl.BlockSpec((1, 2 * VB), lambda p, i: (0, jnp.where(p == 0, i, 0))),
        ],
        out_specs=pl.BlockSpec((1, 2 * VB),
                               lambda p, i: (0, jnp.where(p == 0, 0, i))),
        out_shape=jax.ShapeDtypeStruct((1, VOCAB), jnp.float32),
        scratch_shapes=[pltpu.VMEM((G, 2 * VB), jnp.float32),
                        pltpu.SMEM((1,), jnp.float32),
                        pltpu.SMEM((1,), jnp.float32)],
    )(h, W2, W2, b2.reshape(1, VOCAB))

    return log_probs
